# Initial kernel scaffold; baseline (speedup 1.0000x reference)
import jax
import jax.numpy as jnp
from jax.experimental import pallas as pl


def kernel(x, edge_index, edge_attr, batch, params):
    raise NotImplementedError("write your pallas kernel here")


# speedup vs baseline: 4.6612x; 4.6612x over previous
import functools

import jax
import jax.numpy as jnp
from jax import lax
from jax.experimental import pallas as pl
from jax.experimental.pallas import tpu as pltpu
from jax.experimental.pallas import tpu_sc as plsc

N_NODES = 10000
N_PAD = 10240
N_EDGES = 320000
D_ATOM = 128
D_BOND = 16
N_GRAPH = 32
D_MU = 64
N_STEPS = 4
N_HEADS = 8
D_FF = 512
D_HEAD = 16

RB = 512
CB = 512
N_RB = N_PAD // RB

SC_CORES = 2
SC_TILES = 16
SC_WORKERS = SC_CORES * SC_TILES
E_PER_W = N_EDGES // SC_WORKERS
CHUNK = 80
N_CHUNKS = E_PER_W // CHUNK
ZROWS = N_PAD // SC_TILES

_F32 = jnp.float32


def _sds(shape, dtype=_F32):
    return jax.ShapeDtypeStruct(shape, dtype)


def _edge_sc_body(a_hbm, b_hbm, c_hbm, src_hbm, dst_hbm, zero_hbm, out_hbm,
                  si, di, av, bv, cv, m_sh, sem_a, sem_b, sem_c):
    cid = lax.axis_index("c")
    sid = lax.axis_index("s")

    pltpu.sync_copy(zero_hbm, m_sh.at[pl.ds(sid * ZROWS, ZROWS)])
    plsc.subcore_barrier()

    base = (cid * SC_TILES + sid) * E_PER_W

    def chunk_body(ci, carry):
        off = base + ci * CHUNK
        pltpu.sync_copy(src_hbm.at[pl.ds(off, CHUNK)], si)
        pltpu.sync_copy(dst_hbm.at[pl.ds(off, CHUNK)], di)
        ga = pltpu.async_copy(a_hbm.at[si], av, sem_a)
        gb = pltpu.async_copy(b_hbm.at[di], bv, sem_b)
        gc = pltpu.async_copy(c_hbm.at[pl.ds(off, CHUNK)], cv, sem_c)
        ga.wait()
        gb.wait()
        gc.wait()

        def row_body(r, c2):
            for j in range(D_MU // 16):
                s = pl.ds(j * 16, 16)
                av[r, s] = jnp.maximum(av[r, s] + bv[r, s] + cv[r, s], 0.0)
            return c2

        lax.fori_loop(0, CHUNK, row_body, 0)
        pltpu.sync_copy(av, m_sh.at[di], add=True)
        return carry

    lax.fori_loop(0, N_CHUNKS, chunk_body, 0)
    plsc.subcore_barrier()
    pltpu.sync_copy(m_sh.at[pl.ds(sid * ZROWS, ZROWS)],
                    out_hbm.at[cid, pl.ds(sid * ZROWS, ZROWS)])


def _edge_sc(a, b, c_e, src, dst, zeros):
    call = pl.kernel(
        _edge_sc_body,
        out_type=_sds((SC_CORES, N_PAD, D_MU)),
        mesh=plsc.VectorSubcoreMesh(core_axis_name="c", subcore_axis_name="s"),
        compiler_params=pltpu.CompilerParams(use_tc_tiling_on_sc=False),
        scratch_types=[
            pltpu.VMEM((CHUNK,), jnp.int32),
            pltpu.VMEM((CHUNK,), jnp.int32),
            pltpu.VMEM((CHUNK, D_MU), _F32),
            pltpu.VMEM((CHUNK, D_MU), _F32),
            pltpu.VMEM((CHUNK, D_MU), _F32),
            pltpu.VMEM_SHARED((N_PAD, D_MU), _F32),
            pltpu.SemaphoreType.DMA,
            pltpu.SemaphoreType.DMA,
            pltpu.SemaphoreType.DMA,
        ],
    )
    return call(a, b, c_e, src, dst, zeros)


def _k_edge_const(edge_attr, w_c, b_msg):
    eb = 8000

    def body(ea_ref, w_ref, b_ref, o_ref):
        o_ref[...] = jnp.dot(ea_ref[...], w_ref[...],
                             preferred_element_type=_F32) + b_ref[...]

    return pl.pallas_call(
        body,
        grid=(N_EDGES // eb,),
        in_specs=[pl.BlockSpec((eb, D_BOND), lambda i: (i, 0)),
                  pl.BlockSpec((D_BOND, D_MU), lambda i: (0, 0)),
                  pl.BlockSpec((1, D_MU), lambda i: (0, 0))],
        out_specs=pl.BlockSpec((eb, D_MU), lambda i: (i, 0)),
        out_shape=_sds((N_EDGES, D_MU)),
    )(edge_attr, w_c, b_msg)


def _k_in_proj(x, w, b):
    def body(x_ref, w_ref, b_ref, o_ref):
        o_ref[...] = jnp.dot(x_ref[...], w_ref[...],
                             preferred_element_type=_F32) + b_ref[...]

    return pl.pallas_call(body, out_shape=_sds((N_PAD, D_MU)))(x, w, b)


def _k_ab(h, w_a, w_b):
    def body(h_ref, wa_ref, wb_ref, a_ref, b_ref):
        hh = h_ref[...]
        a_ref[...] = jnp.dot(hh, wa_ref[...], preferred_element_type=_F32)
        b_ref[...] = jnp.dot(hh, wb_ref[...], preferred_element_type=_F32)

    return pl.pallas_call(
        body, out_shape=(_sds((N_PAD, D_MU)), _sds((N_PAD, D_MU)))
    )(h, w_a, w_b)


def _k_gru(mp, h, wz, uz, bz, wr, ur, br, wh, uh, bh):
    def body(mp_ref, h_ref, wz_r, uz_r, bz_r, wr_r, ur_r, br_r,
             wh_r, uh_r, bh_r, o_ref):
        m = mp_ref[0] + mp_ref[1]
        hh = h_ref[...]
        z = jax.nn.sigmoid(jnp.dot(m, wz_r[...], preferred_element_type=_F32)
                           + jnp.dot(hh, uz_r[...], preferred_element_type=_F32)
                           + bz_r[...])
        r = jax.nn.sigmoid(jnp.dot(m, wr_r[...], preferred_element_type=_F32)
                           + jnp.dot(hh, ur_r[...], preferred_element_type=_F32)
                           + br_r[...])
        g = jnp.tanh(jnp.dot(m, wh_r[...], preferred_element_type=_F32)
                     + jnp.dot(r * hh, uh_r[...], preferred_element_type=_F32)
                     + bh_r[...])
        o_ref[...] = (1.0 - z) * hh + z * g

    return pl.pallas_call(body, out_shape=_sds((N_PAD, D_MU)))(
        mp, h, wz, uz, bz, wr, ur, br, wh, uh, bh)


def _k_qkv(h, w_out, b_out, wq, bq, wk, bk, wv, bv):
    def body(h_ref, wo_r, bo_r, wq_r, bq_r, wk_r, bk_r, wv_r, bv_r,
             xn_ref, q_ref, k_ref, v_ref):
        xn = jnp.dot(h_ref[...], wo_r[...], preferred_element_type=_F32) + bo_r[...]
        xn_ref[...] = xn
        q_ref[...] = jnp.dot(xn, wq_r[...], preferred_element_type=_F32) + bq_r[...]
        k_ref[...] = jnp.dot(xn, wk_r[...], preferred_element_type=_F32) + bk_r[...]
        v_ref[...] = jnp.dot(xn, wv_r[...], preferred_element_type=_F32) + bv_r[...]

    return pl.pallas_call(
        body,
        out_shape=(_sds((N_PAD, D_ATOM)), _sds((N_PAD, D_ATOM)),
                   _sds((N_PAD, D_ATOM)), _sds((N_PAD, D_ATOM))),
    )(h, w_out, b_out, wq, bq, wk, bk, wv, bv)


def _ln(t, g, b):
    mu = jnp.mean(t, axis=-1, keepdims=True)
    var = jnp.mean((t - mu) ** 2, axis=-1, keepdims=True)
    return (t - mu) / jnp.sqrt(var + 1e-5) * g + b


def _k_attn_ffn(lo_blk, hi_blk, q, xn, row_lo, row_hi, k, v,
                wo, bo, g1, be1, w1, b1, w2, b2, g2, be2):
    scale = 1.0 / (D_HEAD ** 0.5)

    def body(lo_ref, hi_ref, q_ref, xn_ref, rlo_ref, rhi_ref, k_ref, v_ref,
             wo_r, bo_r, g1_r, be1_r, w1_r, b1_r, w2_r, b2_r, g2_r, be2_r,
             out_ref):
        i = pl.program_id(0)
        blo = lo_ref[i]
        bhi = hi_ref[i]
        qb = q_ref[...]
        rlo = rlo_ref[...]
        rhi = rhi_ref[...]
        col_iota = lax.broadcasted_iota(jnp.int32, (1, CB), 1)

        def cb_body(cb, carry):
            c0 = cb * CB
            kblk = k_ref[pl.ds(c0, CB), :]
            vblk = v_ref[pl.ds(c0, CB), :]
            cols = col_iota + c0
            mask = (cols >= rlo) & (cols < rhi)
            new = []
            for hd in range(N_HEADS):
                sl = slice(hd * D_HEAD, (hd + 1) * D_HEAD)
                m0, l0, acc0 = carry[hd]
                s = lax.dot_general(qb[:, sl], kblk[:, sl],
                                    (((1,), (1,)), ((), ())),
                                    preferred_element_type=_F32) * scale
                s = jnp.where(mask, s, -1e9)
                m1 = jnp.maximum(m0, jnp.max(s, axis=1, keepdims=True))
                p = jnp.exp(s - m1)
                alpha = jnp.exp(m0 - m1)
                l1 = l0 * alpha + jnp.sum(p, axis=1, keepdims=True)
                acc1 = acc0 * alpha + lax.dot_general(
                    p, vblk[:, sl], (((1,), (0,)), ((), ())),
                    preferred_element_type=_F32)
                new.append((m1, l1, acc1))
            return tuple(new)

        init = tuple((jnp.full((RB, 1), -1e30, _F32),
                      jnp.zeros((RB, 1), _F32),
                      jnp.zeros((RB, D_HEAD), _F32)) for _ in range(N_HEADS))
        fin = lax.fori_loop(blo, bhi, cb_body, init)
        o = jnp.concatenate([fin[hd][2] / fin[hd][1] for hd in range(N_HEADS)],
                            axis=1)
        o = jnp.dot(o, wo_r[...], preferred_element_type=_F32) + bo_r[...]
        h1 = _ln(xn_ref[...] + o, g1_r[...], be1_r[...])
        f = jnp.maximum(jnp.dot(h1, w1_r[...], preferred_element_type=_F32)
                        + b1_r[...], 0.0)
        f = jnp.dot(f, w2_r[...], preferred_element_type=_F32) + b2_r[...]
        out_ref[...] = _ln(h1 + f, g2_r[...], be2_r[...])

    smem = pl.BlockSpec(memory_space=pltpu.SMEM)
    full = lambda shape: pl.BlockSpec(shape, lambda i: tuple(0 for _ in shape))
    blk = lambda shape: pl.BlockSpec(shape, lambda i: (i, 0))
    return pl.pallas_call(
        body,
        grid=(N_RB,),
        in_specs=[smem, smem,
                  blk((RB, D_ATOM)), blk((RB, D_ATOM)),
                  blk((RB, 1)), blk((RB, 1)),
                  full((N_PAD, D_ATOM)), full((N_PAD, D_ATOM)),
                  full((D_ATOM, D_ATOM)), full((1, D_ATOM)),
                  full((1, D_ATOM)), full((1, D_ATOM)),
                  full((D_ATOM, D_FF)), full((1, D_FF)),
                  full((D_FF, D_ATOM)), full((1, D_ATOM)),
                  full((1, D_ATOM)), full((1, D_ATOM))],
        out_specs=blk((RB, D_ATOM)),
        out_shape=_sds((N_PAD, D_ATOM)),
    )(lo_blk, hi_blk, q, xn, row_lo, row_hi, k, v,
      wo, bo, g1, be1, w1, b1, w2, b2, g2, be2)


def _k_pool_head(h2, batch_col, counts, wd1, bd1, wd2, bd2):
    def body(h2_ref, bat_ref, cnt_ref, wd1_r, bd1_r, wd2_r, bd2_r,
             out_ref, acc_ref):
        i = pl.program_id(0)
        g_ids = lax.broadcasted_iota(jnp.int32, (1, N_GRAPH), 1)
        ind = (bat_ref[...] == g_ids).astype(_F32)
        part = lax.dot_general(ind, h2_ref[...], (((0,), (0,)), ((), ())),
                               preferred_element_type=_F32)

        @pl.when(i == 0)
        def _():
            acc_ref[...] = part

        @pl.when(i > 0)
        def _():
            acc_ref[...] = acc_ref[...] + part

        @pl.when(i == N_RB - 1)
        def _():
            pooled = acc_ref[...] / cnt_ref[...]
            zz = jnp.maximum(jnp.dot(pooled, wd1_r[...],
                                     preferred_element_type=_F32) + bd1_r[...],
                             0.0)
            out_ref[...] = jax.nn.sigmoid(
                jnp.dot(zz, wd2_r[...], preferred_element_type=_F32) + bd2_r[...])

    full = lambda shape: pl.BlockSpec(shape, lambda i: tuple(0 for _ in shape))
    return pl.pallas_call(
        body,
        grid=(N_RB,),
        in_specs=[pl.BlockSpec((RB, D_ATOM), lambda i: (i, 0)),
                  pl.BlockSpec((RB, 1), lambda i: (i, 0)),
                  full((N_GRAPH, 1)),
                  full((D_ATOM, D_FF)), full((1, D_FF)),
                  full((D_FF, 1)), full((1, 1))],
        out_specs=pl.BlockSpec((N_GRAPH, 1), lambda i: (0, 0)),
        out_shape=_sds((N_GRAPH, 1)),
        scratch_shapes=[pltpu.VMEM((N_GRAPH, D_ATOM), _F32)],
    )(h2, batch_col, counts, wd1, bd1, wd2, bd2)


def kernel(x, edge_index, edge_attr, batch, params):
    p = params
    src = edge_index[0]
    dst = edge_index[1]

    x_pad = jnp.pad(x, ((0, N_PAD - N_NODES), (0, 0)))
    batch_pad = jnp.concatenate(
        [batch, jnp.full((N_PAD - N_NODES,), N_GRAPH, jnp.int32)])
    counts_ext = jnp.bincount(batch_pad, length=N_GRAPH + 1)
    offsets = jnp.concatenate(
        [jnp.zeros((1,), jnp.int32),
         jnp.cumsum(counts_ext).astype(jnp.int32)])
    row_lo = offsets[batch_pad][:, None]
    row_hi = offsets[batch_pad + 1][:, None]
    lo_blk = (row_lo[::RB, 0] // CB).astype(jnp.int32)
    hi_blk = ((row_hi[RB - 1::RB, 0] + CB - 1) // CB).astype(jnp.int32)
    counts = jnp.maximum(jnp.bincount(batch, length=N_GRAPH), 1)
    counts = counts.astype(_F32)[:, None]

    wmsg = p['W_msg']
    w_a, w_b, w_c = wmsg[:D_MU], wmsg[D_MU:2 * D_MU], wmsg[2 * D_MU:]
    zeros = jnp.zeros((ZROWS, D_MU), _F32)

    c_e = _k_edge_const(edge_attr, w_c, p['b_msg'][None, :])
    h = _k_in_proj(x_pad, p['W_in'], p['b_in'][None, :])
    for _ in range(N_STEPS):
        a, bgt = _k_ab(h, w_a, w_b)
        mp = _edge_sc(a, bgt, c_e, src, dst, zeros)
        h = _k_gru(mp, h,
                   p['Wz'], p['Uz'], p['bz'][None, :],
                   p['Wr'], p['Ur'], p['br'][None, :],
                   p['Wh'], p['Uh'], p['bh'][None, :])

    xn, q, k, v = _k_qkv(h, p['W_out'], p['b_out'][None, :],
                         p['Wqa'], p['bqa'][None, :],
                         p['Wka'], p['bka'][None, :],
                         p['Wva'], p['bva'][None, :])
    h2 = _k_attn_ffn(lo_blk, hi_blk, q, xn, row_lo, row_hi, k, v,
                     p['Woa'], p['boa'][None, :],
                     p['ln1_g'][None, :], p['ln1_b'][None, :],
                     p['W1'], p['b1'][None, :],
                     p['W2'], p['b2'][None, :],
                     p['ln2_g'][None, :], p['ln2_b'][None, :])
    return _k_pool_head(h2, batch_pad[:, None], counts,
                        p['Wd1'], p['bd1'][None, :],
                        p['Wd2'], p['bd2'][None, :])


# speedup vs baseline: 7.1676x; 1.5377x over previous
import functools

import jax
import jax.numpy as jnp
from jax import lax
from jax.experimental import pallas as pl
from jax.experimental.pallas import tpu as pltpu
from jax.experimental.pallas import tpu_sc as plsc

N_NODES = 10000
N_PAD = 10240
N_EDGES = 320000
D_ATOM = 128
D_BOND = 16
N_GRAPH = 32
D_MU = 64
N_STEPS = 4
N_HEADS = 8
D_FF = 512
D_HEAD = 16

RB = 512
CB = 512
N_RB = N_PAD // RB

SC_CORES = 2
SC_TILES = 16
SC_WORKERS = SC_CORES * SC_TILES
E_PER_W = N_EDGES // SC_WORKERS
CHUNK = 80
N_CHUNKS = E_PER_W // CHUNK
ZROWS = N_PAD // SC_TILES

_F32 = jnp.float32


def _sds(shape, dtype=_F32):
    return jax.ShapeDtypeStruct(shape, dtype)


def _edge_sc_body(a_hbm, b_hbm, c_hbm, si3_hbm, di3_hbm, zero_hbm, out_hbm,
                  si2, di2, av0, bv0, cv0, rv0, av1, bv1, cv1, rv1,
                  m_sh, sg0, sg1, ss0, ss1):
    cid = lax.axis_index("c")
    sid = lax.axis_index("s")
    wid = cid * SC_TILES + sid
    base = wid * E_PER_W

    pltpu.sync_copy(zero_hbm, m_sh.at[pl.ds(sid * ZROWS, ZROWS)])
    pltpu.sync_copy(si3_hbm.at[wid], si2)
    pltpu.sync_copy(di3_hbm.at[wid], di2)
    plsc.subcore_barrier()

    b0 = (av0, bv0, cv0, rv0, sg0, ss0)
    b1 = (av1, bv1, cv1, rv1, sg1, ss1)

    def startg(ci, b):
        av, bv, cv, _, sg, _ = b
        pltpu.async_copy(a_hbm.at[si2.at[ci]], av, sg)
        pltpu.async_copy(b_hbm.at[di2.at[ci]], bv, sg)
        pltpu.async_copy(c_hbm.at[pl.ds(base + ci * CHUNK, CHUNK)], cv, sg)

    def waitg(b):
        av, bv, cv, _, sg, _ = b
        for dst in (av, bv, cv):
            pltpu.make_async_copy(a_hbm.at[pl.ds(0, CHUNK)], dst, sg).wait()

    def waitsc(b):
        rv, ss = b[3], b[5]
        pltpu.make_async_copy(a_hbm.at[pl.ds(0, CHUNK)], rv, ss).wait()

    def do_slot(ci, b, refill_ci, scwait):
        av, bv, cv, rv, sg, ss = b
        waitg(b)
        if scwait:
            waitsc(b)

        @plsc.parallel_loop(0, CHUNK, unroll=4)
        def _(r):
            for j in range(D_MU // 16):
                s = pl.ds(j * 16, 16)
                rv[r, s] = jnp.maximum(av[r, s] + bv[r, s] + cv[r, s], 0.0)

        if refill_ci is not None:
            startg(refill_ci, b)
        pltpu.async_copy(rv, m_sh.at[di2.at[ci]], ss, add=True)

    startg(0, b0)
    startg(1, b1)
    do_slot(0, b0, 2, False)
    do_slot(1, b1, 3, False)

    def loop_body(g, carry):
        c0 = 2 * g + 2
        do_slot(c0, b0, c0 + 2, True)
        do_slot(c0 + 1, b1, c0 + 3, True)
        return carry

    lax.fori_loop(0, (N_CHUNKS - 5) // 2, loop_body, 0)
    do_slot(N_CHUNKS - 3, b0, N_CHUNKS - 1, True)
    do_slot(N_CHUNKS - 2, b1, None, True)
    do_slot(N_CHUNKS - 1, b0, None, True)
    waitsc(b1)
    waitsc(b0)

    plsc.subcore_barrier()
    pltpu.sync_copy(m_sh.at[pl.ds(sid * ZROWS, ZROWS)],
                    out_hbm.at[cid, pl.ds(sid * ZROWS, ZROWS)])


def _edge_sc(a, b, c_e, si3, di3, zeros):
    call = pl.kernel(
        _edge_sc_body,
        out_type=_sds((SC_CORES, N_PAD, D_MU)),
        mesh=plsc.VectorSubcoreMesh(core_axis_name="c", subcore_axis_name="s"),
        compiler_params=pltpu.CompilerParams(use_tc_tiling_on_sc=False),
        scratch_types=[
            pltpu.VMEM((N_CHUNKS, CHUNK), jnp.int32),
            pltpu.VMEM((N_CHUNKS, CHUNK), jnp.int32),
            pltpu.VMEM((CHUNK, D_MU), _F32),
            pltpu.VMEM((CHUNK, D_MU), _F32),
            pltpu.VMEM((CHUNK, D_MU), _F32),
            pltpu.VMEM((CHUNK, D_MU), _F32),
            pltpu.VMEM((CHUNK, D_MU), _F32),
            pltpu.VMEM((CHUNK, D_MU), _F32),
            pltpu.VMEM((CHUNK, D_MU), _F32),
            pltpu.VMEM((CHUNK, D_MU), _F32),
            pltpu.VMEM_SHARED((N_PAD, D_MU), _F32),
            pltpu.SemaphoreType.DMA,
            pltpu.SemaphoreType.DMA,
            pltpu.SemaphoreType.DMA,
            pltpu.SemaphoreType.DMA,
        ],
    )
    return call(a, b, c_e, si3, di3, zeros)


def _k_edge_const(edge_attr, w_c, b_msg):
    eb = 8000

    def body(ea_ref, w_ref, b_ref, o_ref):
        o_ref[...] = jnp.dot(ea_ref[...], w_ref[...],
                             preferred_element_type=_F32) + b_ref[...]

    return pl.pallas_call(
        body,
        grid=(N_EDGES // eb,),
        in_specs=[pl.BlockSpec((eb, D_BOND), lambda i: (i, 0)),
                  pl.BlockSpec((D_BOND, D_MU), lambda i: (0, 0)),
                  pl.BlockSpec((1, D_MU), lambda i: (0, 0))],
        out_specs=pl.BlockSpec((eb, D_MU), lambda i: (i, 0)),
        out_shape=_sds((N_EDGES, D_MU)),
    )(edge_attr, w_c, b_msg)


def _k_in_proj(x, w, b):
    def body(x_ref, w_ref, b_ref, o_ref):
        o_ref[...] = jnp.dot(x_ref[...], w_ref[...],
                             preferred_element_type=_F32) + b_ref[...]

    return pl.pallas_call(body, out_shape=_sds((N_PAD, D_MU)))(x, w, b)


def _k_ab(h, w_a, w_b):
    def body(h_ref, wa_ref, wb_ref, a_ref, b_ref):
        hh = h_ref[...]
        a_ref[...] = jnp.dot(hh, wa_ref[...], preferred_element_type=_F32)
        b_ref[...] = jnp.dot(hh, wb_ref[...], preferred_element_type=_F32)

    return pl.pallas_call(
        body, out_shape=(_sds((N_PAD, D_MU)), _sds((N_PAD, D_MU)))
    )(h, w_a, w_b)


def _k_gru(mp, h, wz, uz, bz, wr, ur, br, wh, uh, bh):
    def body(mp_ref, h_ref, wz_r, uz_r, bz_r, wr_r, ur_r, br_r,
             wh_r, uh_r, bh_r, o_ref):
        m = mp_ref[0] + mp_ref[1]
        hh = h_ref[...]
        z = jax.nn.sigmoid(jnp.dot(m, wz_r[...], preferred_element_type=_F32)
                           + jnp.dot(hh, uz_r[...], preferred_element_type=_F32)
                           + bz_r[...])
        r = jax.nn.sigmoid(jnp.dot(m, wr_r[...], preferred_element_type=_F32)
                           + jnp.dot(hh, ur_r[...], preferred_element_type=_F32)
                           + br_r[...])
        g = jnp.tanh(jnp.dot(m, wh_r[...], preferred_element_type=_F32)
                     + jnp.dot(r * hh, uh_r[...], preferred_element_type=_F32)
                     + bh_r[...])
        o_ref[...] = (1.0 - z) * hh + z * g

    return pl.pallas_call(body, out_shape=_sds((N_PAD, D_MU)))(
        mp, h, wz, uz, bz, wr, ur, br, wh, uh, bh)


def _k_qkv(h, w_out, b_out, wq, bq, wk, bk, wv, bv):
    def body(h_ref, wo_r, bo_r, wq_r, bq_r, wk_r, bk_r, wv_r, bv_r,
             xn_ref, q_ref, k_ref, v_ref):
        xn = jnp.dot(h_ref[...], wo_r[...], preferred_element_type=_F32) + bo_r[...]
        xn_ref[...] = xn
        q_ref[...] = jnp.dot(xn, wq_r[...], preferred_element_type=_F32) + bq_r[...]
        k_ref[...] = jnp.dot(xn, wk_r[...], preferred_element_type=_F32) + bk_r[...]
        v_ref[...] = jnp.dot(xn, wv_r[...], preferred_element_type=_F32) + bv_r[...]

    return pl.pallas_call(
        body,
        out_shape=(_sds((N_PAD, D_ATOM)), _sds((N_PAD, D_ATOM)),
                   _sds((N_PAD, D_ATOM)), _sds((N_PAD, D_ATOM))),
    )(h, w_out, b_out, wq, bq, wk, bk, wv, bv)


def _ln(t, g, b):
    mu = jnp.mean(t, axis=-1, keepdims=True)
    var = jnp.mean((t - mu) ** 2, axis=-1, keepdims=True)
    return (t - mu) / jnp.sqrt(var + 1e-5) * g + b


def _k_attn_ffn(lo_blk, hi_blk, q, xn, row_lo, row_hi, k, v,
                wo, bo, g1, be1, w1, b1, w2, b2, g2, be2):
    scale = 1.0 / (D_HEAD ** 0.5)

    def body(lo_ref, hi_ref, q_ref, xn_ref, rlo_ref, rhi_ref, k_ref, v_ref,
             wo_r, bo_r, g1_r, be1_r, w1_r, b1_r, w2_r, b2_r, g2_r, be2_r,
             out_ref):
        i = pl.program_id(0)
        blo = lo_ref[i]
        bhi = hi_ref[i]
        qb = q_ref[...]
        rlo = rlo_ref[...]
        rhi = rhi_ref[...]
        col_iota = lax.broadcasted_iota(jnp.int32, (1, CB), 1)

        def cb_body(cb, carry):
            c0 = cb * CB
            kblk = k_ref[pl.ds(c0, CB), :]
            vblk = v_ref[pl.ds(c0, CB), :]
            cols = col_iota + c0
            mask = (cols >= rlo) & (cols < rhi)
            new = []
            for hd in range(N_HEADS):
                sl = slice(hd * D_HEAD, (hd + 1) * D_HEAD)
                m0, l0, acc0 = carry[hd]
                s = lax.dot_general(qb[:, sl], kblk[:, sl],
                                    (((1,), (1,)), ((), ())),
                                    preferred_element_type=_F32) * scale
                s = jnp.where(mask, s, -1e9)
                m1 = jnp.maximum(m0, jnp.max(s, axis=1, keepdims=True))
                p = jnp.exp(s - m1)
                alpha = jnp.exp(m0 - m1)
                l1 = l0 * alpha + jnp.sum(p, axis=1, keepdims=True)
                acc1 = acc0 * alpha + lax.dot_general(
                    p, vblk[:, sl], (((1,), (0,)), ((), ())),
                    preferred_element_type=_F32)
                new.append((m1, l1, acc1))
            return tuple(new)

        init = tuple((jnp.full((RB, 1), -1e30, _F32),
                      jnp.zeros((RB, 1), _F32),
                      jnp.zeros((RB, D_HEAD), _F32)) for _ in range(N_HEADS))
        fin = lax.fori_loop(blo, bhi, cb_body, init)
        o = jnp.concatenate([fin[hd][2] / fin[hd][1] for hd in range(N_HEADS)],
                            axis=1)
        o = jnp.dot(o, wo_r[...], preferred_element_type=_F32) + bo_r[...]
        h1 = _ln(xn_ref[...] + o, g1_r[...], be1_r[...])
        f = jnp.maximum(jnp.dot(h1, w1_r[...], preferred_element_type=_F32)
                        + b1_r[...], 0.0)
        f = jnp.dot(f, w2_r[...], preferred_element_type=_F32) + b2_r[...]
        out_ref[...] = _ln(h1 + f, g2_r[...], be2_r[...])

    smem = pl.BlockSpec(memory_space=pltpu.SMEM)
    full = lambda shape: pl.BlockSpec(shape, lambda i: tuple(0 for _ in shape))
    blk = lambda shape: pl.BlockSpec(shape, lambda i: (i, 0))
    return pl.pallas_call(
        body,
        grid=(N_RB,),
        in_specs=[smem, smem,
                  blk((RB, D_ATOM)), blk((RB, D_ATOM)),
                  blk((RB, 1)), blk((RB, 1)),
                  full((N_PAD, D_ATOM)), full((N_PAD, D_ATOM)),
                  full((D_ATOM, D_ATOM)), full((1, D_ATOM)),
                  full((1, D_ATOM)), full((1, D_ATOM)),
                  full((D_ATOM, D_FF)), full((1, D_FF)),
                  full((D_FF, D_ATOM)), full((1, D_ATOM)),
                  full((1, D_ATOM)), full((1, D_ATOM))],
        out_specs=blk((RB, D_ATOM)),
        out_shape=_sds((N_PAD, D_ATOM)),
    )(lo_blk, hi_blk, q, xn, row_lo, row_hi, k, v,
      wo, bo, g1, be1, w1, b1, w2, b2, g2, be2)


def _k_pool_head(h2, batch_col, counts, wd1, bd1, wd2, bd2):
    def body(h2_ref, bat_ref, cnt_ref, wd1_r, bd1_r, wd2_r, bd2_r,
             out_ref, acc_ref):
        i = pl.program_id(0)
        g_ids = lax.broadcasted_iota(jnp.int32, (1, N_GRAPH), 1)
        ind = (bat_ref[...] == g_ids).astype(_F32)
        part = lax.dot_general(ind, h2_ref[...], (((0,), (0,)), ((), ())),
                               preferred_element_type=_F32)

        @pl.when(i == 0)
        def _():
            acc_ref[...] = part

        @pl.when(i > 0)
        def _():
            acc_ref[...] = acc_ref[...] + part

        @pl.when(i == N_RB - 1)
        def _():
            pooled = acc_ref[...] / cnt_ref[...]
            zz = jnp.maximum(jnp.dot(pooled, wd1_r[...],
                                     preferred_element_type=_F32) + bd1_r[...],
                             0.0)
            out_ref[...] = jax.nn.sigmoid(
                jnp.dot(zz, wd2_r[...], preferred_element_type=_F32) + bd2_r[...])

    full = lambda shape: pl.BlockSpec(shape, lambda i: tuple(0 for _ in shape))
    return pl.pallas_call(
        body,
        grid=(N_RB,),
        in_specs=[pl.BlockSpec((RB, D_ATOM), lambda i: (i, 0)),
                  pl.BlockSpec((RB, 1), lambda i: (i, 0)),
                  full((N_GRAPH, 1)),
                  full((D_ATOM, D_FF)), full((1, D_FF)),
                  full((D_FF, 1)), full((1, 1))],
        out_specs=pl.BlockSpec((N_GRAPH, 1), lambda i: (0, 0)),
        out_shape=_sds((N_GRAPH, 1)),
        scratch_shapes=[pltpu.VMEM((N_GRAPH, D_ATOM), _F32)],
    )(h2, batch_col, counts, wd1, bd1, wd2, bd2)


def kernel(x, edge_index, edge_attr, batch, params):
    p = params
    si3 = edge_index[0].reshape(SC_WORKERS, N_CHUNKS, CHUNK)
    di3 = edge_index[1].reshape(SC_WORKERS, N_CHUNKS, CHUNK)

    x_pad = jnp.pad(x, ((0, N_PAD - N_NODES), (0, 0)))
    batch_pad = jnp.concatenate(
        [batch, jnp.full((N_PAD - N_NODES,), N_GRAPH, jnp.int32)])
    counts_ext = jnp.bincount(batch_pad, length=N_GRAPH + 1)
    offsets = jnp.concatenate(
        [jnp.zeros((1,), jnp.int32),
         jnp.cumsum(counts_ext).astype(jnp.int32)])
    row_lo = offsets[batch_pad][:, None]
    row_hi = offsets[batch_pad + 1][:, None]
    lo_blk = (row_lo[::RB, 0] // CB).astype(jnp.int32)
    hi_blk = ((row_hi[RB - 1::RB, 0] + CB - 1) // CB).astype(jnp.int32)
    counts = jnp.maximum(jnp.bincount(batch, length=N_GRAPH), 1)
    counts = counts.astype(_F32)[:, None]

    wmsg = p['W_msg']
    w_a, w_b, w_c = wmsg[:D_MU], wmsg[D_MU:2 * D_MU], wmsg[2 * D_MU:]
    zeros = jnp.zeros((ZROWS, D_MU), _F32)

    c_e = _k_edge_const(edge_attr, w_c, p['b_msg'][None, :])
    h = _k_in_proj(x_pad, p['W_in'], p['b_in'][None, :])
    for _ in range(N_STEPS):
        a, bgt = _k_ab(h, w_a, w_b)
        mp = _edge_sc(a, bgt, c_e, si3, di3, zeros)
        h = _k_gru(mp, h,
                   p['Wz'], p['Uz'], p['bz'][None, :],
                   p['Wr'], p['Ur'], p['br'][None, :],
                   p['Wh'], p['Uh'], p['bh'][None, :])

    xn, q, k, v = _k_qkv(h, p['W_out'], p['b_out'][None, :],
                         p['Wqa'], p['bqa'][None, :],
                         p['Wka'], p['bka'][None, :],
                         p['Wva'], p['bva'][None, :])
    h2 = _k_attn_ffn(lo_blk, hi_blk, q, xn, row_lo, row_hi, k, v,
                     p['Woa'], p['boa'][None, :],
                     p['ln1_g'][None, :], p['ln1_b'][None, :],
                     p['W1'], p['b1'][None, :],
                     p['W2'], p['b2'][None, :],
                     p['ln2_g'][None, :], p['ln2_b'][None, :])
    return _k_pool_head(h2, batch_pad[:, None], counts,
                        p['Wd1'], p['bd1'][None, :],
                        p['Wd2'], p['bd2'][None, :])
